# baseline (speedup 1.0000x reference)
import functools

import jax
import jax.numpy as jnp
from jax import lax
from jax.experimental import pallas as pl
from jax.experimental.pallas import tpu as pltpu
from jax.experimental.pallas import tpu_sc as plsc

_NBUF = 4
_RC = 128


def _embed_concat(X_num, idx_flat, tab_flat, *, B, F, V, D, NN):
    OUT_W = F * D + NN
    info = plsc.get_sparse_core_info()
    NC, NS = info.num_cores, info.num_subcores
    NW = NC * NS
    BW = B // NW
    NCHK = BW // _RC
    T = F * NCHK

    mesh = plsc.VectorSubcoreMesh(core_axis_name="c", subcore_axis_name="s")

    @functools.partial(
        pl.kernel,
        out_type=jax.ShapeDtypeStruct((B, OUT_W), jnp.float32),
        mesh=mesh,
        compiler_params=pltpu.CompilerParams(use_tc_tiling_on_sc=False),
        scratch_types=[
            pltpu.VMEM((F * BW,), jnp.int32),
            [pltpu.VMEM((_RC, D), jnp.float32)] * _NBUF,
            pltpu.VMEM((BW, NN), jnp.float32),
            [pltpu.SemaphoreType.DMA] * _NBUF,
            [pltpu.SemaphoreType.DMA] * _NBUF,
            pltpu.SemaphoreType.DMA,
            pltpu.SemaphoreType.DMA,
        ],
    )
    def run(idx_hbm, xnum_hbm, tab_hbm, out_hbm,
            idxv, gbufs, xnum_v, gsems, wsems, isem, nsem):
        wid = lax.axis_index("s") * NC + lax.axis_index("c")
        wbase = pl.multiple_of(wid * BW, BW)

        def gather(t):
            f, c = t // NCHK, t % NCHK
            b = t % _NBUF
            return pltpu.async_copy(
                tab_hbm.at[idxv.at[pl.ds(f * BW + c * _RC, _RC)]],
                gbufs[b],
                gsems[b],
            )

        def write(t):
            f, c = t // NCHK, t % NCHK
            b = t % _NBUF
            return pltpu.async_copy(
                gbufs[b],
                out_hbm.at[pl.ds(wbase + c * _RC, _RC), pl.ds(f * D, D)],
                wsems[b],
            )

        icps = [
            pltpu.async_copy(
                idx_hbm.at[pl.ds(f * B + wbase, BW)],
                idxv.at[pl.ds(f * BW, BW)],
                isem,
            )
            for f in range(F)
        ]
        ncp = pltpu.async_copy(xnum_hbm.at[pl.ds(wbase, BW)], xnum_v, nsem)
        for cp in icps:
            cp.wait()

        g, w = {}, {}
        for t in range(_NBUF - 1):
            g[t] = gather(t)
        ncp.wait()
        nw = pltpu.async_copy(
            xnum_v, out_hbm.at[pl.ds(wbase, BW), pl.ds(F * D, NN)], nsem
        )
        for t in range(T):
            nt = t + _NBUF - 1
            if nt < T:
                if nt >= _NBUF:
                    w[nt - _NBUF].wait()
                g[nt] = gather(nt)
            g[t].wait()
            w[t] = write(t)
        for t in range(max(0, T - _NBUF), T):
            w[t].wait()
        nw.wait()

    return run(idx_flat, X_num, tab_flat)


def kernel(X_num, X_cat, tables):
    B, NN = X_num.shape
    _, F = X_cat.shape
    _, V, D = tables.shape
    idx_flat = (X_cat.astype(jnp.int32).T
                + (jnp.arange(F, dtype=jnp.int32) * V)[:, None]).reshape(F * B)
    tab_flat = tables.reshape(F * V, D)
    return _embed_concat(X_num, idx_flat, tab_flat, B=B, F=F, V=V, D=D, NN=NN)
